# baseline (speedup 1.0000x reference)
import functools

import jax
import jax.numpy as jnp
from jax import lax
from jax.experimental import pallas as pl
from jax.experimental.pallas import tpu as pltpu
from jax.experimental.pallas import tpu_sc as plsc

N = 10000
G = 8
E = 160000
FIN = 128
HG = 32
HR = 128
HD = 128
FNWP = 16
HOR = 4

NC = 2
NS = 16
GPC = G // NC
PW = GPC * HG

CHUNK = 128
CPT = 80
EPT = CHUNK * CPT
EPAD = EPT * NS
NPAD = N + 16
RT = 1000
NT_OUT = N // RT
RZ = NPAD // NS

HALF = CPT // 2
R = 4

NB = 5
RB = N // NB

_mesh = plsc.VectorSubcoreMesh(
    core_axis_name="c", subcore_axis_name="s", num_cores=NC, num_subcores=NS)

_sc_params = pltpu.CompilerParams(use_tc_tiling_on_sc=False)


@functools.partial(
    pl.kernel,
    out_type=jax.ShapeDtypeStruct((N, 16), jnp.float32),
    mesh=_mesh,
    compiler_params=_sc_params,
    scratch_types=[
        pltpu.VMEM((CPT, CHUNK), jnp.int32),
        pltpu.VMEM((CHUNK, 16), jnp.float32),
        pltpu.VMEM((RZ, 16), jnp.float32),
        pltpu.VMEM_SHARED((NPAD, 16), jnp.float32),
    ],
)
def _deg_kernel(dst_hbm, deg_hbm, dst_v, ones_v, zero_v, deg_sh):
    c = lax.axis_index("c")
    s = lax.axis_index("s")
    pltpu.sync_copy(dst_hbm.at[s], dst_v)

    def _fill(i, carry):
        ones_v[i, :] = jnp.ones((16,), jnp.float32)
        return carry
    lax.fori_loop(0, CHUNK, _fill, 0)

    def _zfill(i, carry):
        zero_v[i, :] = jnp.zeros((16,), jnp.float32)
        return carry
    lax.fori_loop(0, RZ, _zfill, 0)

    pltpu.sync_copy(zero_v, deg_sh.at[pl.ds(s * RZ, RZ)])
    plsc.subcore_barrier()

    def _body(j, carry):
        pltpu.sync_copy(ones_v, deg_sh.at[dst_v.at[j]], add=True)
        return carry
    lax.fori_loop(0, CPT, _body, 0)
    plsc.subcore_barrier()

    @pl.when(jnp.logical_and(c == 0, s < NT_OUT))
    def _():
        pltpu.sync_copy(deg_sh.at[pl.ds(s * RT, RT)],
                        deg_hbm.at[pl.ds(s * RT, RT)])


@functools.partial(
    pl.kernel,
    out_type=jax.ShapeDtypeStruct((NC, N, PW), jnp.bfloat16),
    mesh=_mesh,
    compiler_params=_sc_params,
    scratch_types=[
        pltpu.VMEM((HALF, CHUNK), jnp.int32),
        pltpu.VMEM((HALF, CHUNK), jnp.int32),
        pltpu.VMEM((R, CHUNK, PW), jnp.bfloat16),
        pltpu.VMEM_SHARED((NPAD, PW), jnp.bfloat16),
        pltpu.SemaphoreType.DMA,
        pltpu.SemaphoreType.DMA,
        pltpu.SemaphoreType.DMA,
        pltpu.SemaphoreType.DMA,
        pltpu.SemaphoreType.DMA,
        pltpu.SemaphoreType.DMA,
        pltpu.SemaphoreType.DMA,
        pltpu.SemaphoreType.DMA,
    ],
)
def _conv_kernel(hs_hbm, src_hbm, dst_hbm, out_hbm,
                 dst_v, idx_v, rows_v, acc_sh,
                 sem_g0, sem_g1, sem_g2, sem_g3,
                 sem_s0, sem_s1, sem_s2, sem_s3):
    c = lax.axis_index("c")
    s = lax.axis_index("s")
    sem_g = (sem_g0, sem_g1, sem_g2, sem_g3)
    sem_s = (sem_s0, sem_s1, sem_s2, sem_s3)
    off = c * N

    def _zfill(i, carry):
        for k in range(PW // 32):
            rows_v[0, i, pl.ds(k * 32, 32)] = jnp.zeros((32,), jnp.bfloat16)
        return carry
    lax.fori_loop(0, CHUNK, _zfill, 0)
    for m in range(4):
        pltpu.sync_copy(rows_v.at[0],
                        acc_sh.at[pl.ds(s * RZ + m * CHUNK, CHUNK)])
    pltpu.sync_copy(rows_v.at[0, pl.ds(0, RZ - 4 * CHUNK)],
                    acc_sh.at[pl.ds(s * RZ + 4 * CHUNK, RZ - 4 * CHUNK)])
    plsc.subcore_barrier()

    def _gather(j, slot):
        pltpu.async_copy(
            hs_hbm.at[idx_v.at[j]], rows_v.at[slot], sem_g[slot])

    def _gather_wait(j, slot):
        pltpu.make_async_copy(
            hs_hbm.at[idx_v.at[j]], rows_v.at[slot], sem_g[slot]).wait()

    def _scatter(j, slot):
        pltpu.async_copy(
            rows_v.at[slot], acc_sh.at[dst_v.at[j]], sem_s[slot], add=True)

    def _scatter_wait(j, slot):
        pltpu.make_async_copy(
            rows_v.at[slot], acc_sh.at[dst_v.at[j]], sem_s[slot]).wait()

    for half in range(2):
        base = half * HALF
        pltpu.sync_copy(src_hbm.at[s, pl.ds(base, HALF)], idx_v)
        pltpu.sync_copy(dst_hbm.at[s, pl.ds(base, HALF)], dst_v)

        def _idxfill(j, c2):
            for k in range(CHUNK // 16):
                idx_v[j, pl.ds(k * 16, 16)] = (
                    idx_v[j, pl.ds(k * 16, 16)] + off)
            return c2
        lax.fori_loop(0, HALF, _idxfill, 0)

        for r in range(R - 1):
            _gather(r, r)

        def _grp_body(q, c2):
            for r in range(R):
                j = q * R + r
                nslot = (r + R - 1) % R
                _gather_wait(j, r)
                _scatter(j, r)
                @pl.when(j >= 1)
                def _():
                    _scatter_wait(j - 1, nslot)

                @pl.when(j + R - 1 < HALF)
                def _():
                    _gather(j + R - 1, nslot)
            return c2
        lax.fori_loop(0, HALF // R, _grp_body, 0)
        _scatter_wait(HALF - 1, R - 1)

    plsc.subcore_barrier()

    @pl.when(s < NT_OUT)
    def _():
        pltpu.sync_copy(acc_sh.at[pl.ds(s * RT, RT)],
                        out_hbm.at[c, pl.ds(s * RT, RT)])


def _xw_scale_body(x_ref, w_ref, deg_ref, o_ref):
    dinv = lax.rsqrt(deg_ref[:, 0:1] + 1.0)
    w = w_ref[...]
    parts = [
        jnp.dot(x_ref[gl], w, preferred_element_type=jnp.float32)
        for gl in range(GPC)
    ]
    o_ref[...] = (jnp.concatenate(parts, axis=1) * dinv).astype(jnp.bfloat16)


def _mid_body(acc_ref, hs_ref, deg_ref, b1_ref, w2_ref, o_ref):
    dinv = lax.rsqrt(deg_ref[:, 0:1] + 1.0)
    b1 = b1_ref[...]
    w2 = w2_ref[...]
    acc = acc_ref[...].astype(jnp.float32)
    hs = hs_ref[...].astype(jnp.float32)
    parts = []
    for gl in range(GPC):
        lo, hi = gl * HG, (gl + 1) * HG
        h1 = jnp.maximum(dinv * (acc[:, lo:hi] + hs[:, lo:hi]) + b1, 0.0)
        parts.append(jnp.dot(h1, w2, preferred_element_type=jnp.float32))
    o_ref[...] = (jnp.concatenate(parts, axis=1) * dinv).astype(jnp.bfloat16)


def _final_body(acc_ref, hs_ref, deg_ref, b2_ref, wih_ref, whh_ref,
                bih_ref, bhh_ref, xf_ref, wd1h_ref, wd1f_ref, bd1_ref,
                wd2_ref, bd2_ref, y_ref):
    dinv = lax.rsqrt(deg_ref[:, 0:1] + 1.0)
    b2 = b2_ref[...]
    wih = wih_ref[...].astype(jnp.bfloat16)
    whh = whh_ref[...].astype(jnp.bfloat16)
    bih = bih_ref[...]
    bhh = bhh_ref[...]
    h = jnp.zeros((RB, HR), jnp.float32)
    for t in range(G):
        cc, gl = t // GPC, t % GPC
        sl = pl.ds(gl * HG, HG)
        xt = dinv * (acc_ref[cc, :, sl].astype(jnp.float32)
                     + hs_ref[cc, :, sl].astype(jnp.float32)) + b2
        gi = jnp.dot(xt.astype(jnp.bfloat16), wih,
                     preferred_element_type=jnp.float32) + bih
        gh = jnp.dot(h.astype(jnp.bfloat16), whh,
                     preferred_element_type=jnp.float32) + bhh
        r = jax.nn.sigmoid(gi[:, 0:HR] + gh[:, 0:HR])
        z = jax.nn.sigmoid(gi[:, HR:2 * HR] + gh[:, HR:2 * HR])
        n = jnp.tanh(gi[:, 2 * HR:] + r * gh[:, 2 * HR:])
        h = (1.0 - z) * n + z * h
    hp = jnp.dot(h.astype(jnp.bfloat16), wd1h_ref[...].astype(jnp.bfloat16),
                 preferred_element_type=jnp.float32)
    wd1f = wd1f_ref[...]
    bd1 = bd1_ref[...]
    wd2 = wd2_ref[...].astype(jnp.bfloat16)
    bd2 = bd2_ref[...]
    for t in range(HOR):
        zz = jnp.maximum(
            hp + jnp.dot(xf_ref[t], wd1f, preferred_element_type=jnp.float32)
            + bd1, 0.0)
        y_ref[t] = jnp.dot(zz.astype(jnp.bfloat16), wd2,
                           preferred_element_type=jnp.float32) + bd2


def _xw_scale(x_g, w1, deg):
    return pl.pallas_call(
        _xw_scale_body,
        grid=(NC, NB),
        in_specs=[
            pl.BlockSpec((GPC, RB, FIN), lambda cb, nb: (cb, nb, 0)),
            pl.BlockSpec((FIN, HG), lambda cb, nb: (0, 0)),
            pl.BlockSpec((RB, 16), lambda cb, nb: (nb, 0)),
        ],
        out_specs=pl.BlockSpec((RB, PW), lambda cb, nb: (cb * NB + nb, 0)),
        out_shape=jax.ShapeDtypeStruct((NC * N, PW), jnp.bfloat16),
    )(x_g, w1, deg)


def _mid(acc1, hs1, deg, b1, w2):
    return pl.pallas_call(
        _mid_body,
        grid=(NC * NB,),
        in_specs=[
            pl.BlockSpec((RB, PW), lambda i: (i, 0)),
            pl.BlockSpec((RB, PW), lambda i: (i, 0)),
            pl.BlockSpec((RB, 16), lambda i: (i % NB, 0)),
            pl.BlockSpec((1, HG), lambda i: (0, 0)),
            pl.BlockSpec((HG, HG), lambda i: (0, 0)),
        ],
        out_specs=pl.BlockSpec((RB, PW), lambda i: (i, 0)),
        out_shape=jax.ShapeDtypeStruct((NC * N, PW), jnp.bfloat16),
    )(acc1, hs1, deg, b1, w2)


def _final(acc2, hs2, deg, b2, wih_t, whh_t, bih, bhh, xf, wd1h, wd1f,
           bd1, wd2, bd2):
    return pl.pallas_call(
        _final_body,
        grid=(NB,),
        in_specs=[
            pl.BlockSpec((NC, RB, PW), lambda i: (0, i, 0)),
            pl.BlockSpec((NC, RB, PW), lambda i: (0, i, 0)),
            pl.BlockSpec((RB, 16), lambda i: (i, 0)),
            pl.BlockSpec((1, HG), lambda i: (0, 0)),
            pl.BlockSpec((HG, 3 * HR), lambda i: (0, 0)),
            pl.BlockSpec((HR, 3 * HR), lambda i: (0, 0)),
            pl.BlockSpec((1, 3 * HR), lambda i: (0, 0)),
            pl.BlockSpec((1, 3 * HR), lambda i: (0, 0)),
            pl.BlockSpec((HOR, RB, FNWP), lambda i: (0, i, 0)),
            pl.BlockSpec((HR, HD), lambda i: (0, 0)),
            pl.BlockSpec((FNWP, HD), lambda i: (0, 0)),
            pl.BlockSpec((1, HD), lambda i: (0, 0)),
            pl.BlockSpec((HD, 1), lambda i: (0, 0)),
            pl.BlockSpec((1, 1), lambda i: (0, 0)),
        ],
        out_specs=pl.BlockSpec((HOR, RB, 1), lambda i: (0, i, 0)),
        out_shape=jax.ShapeDtypeStruct((HOR, N, 1), jnp.float32),
    )(acc2, hs2, deg, b2, wih_t, whh_t, bih, bhh, xf, wd1h, wd1f,
      bd1, wd2, bd2)


def kernel(X_seq, X_fut_seq, edge_index, W1, b1, W2, b2, W_ih, W_hh,
           b_ih, b_hh, Wd1, bd1, Wd2, bd2):
    src = edge_index[0]
    dst = edge_index[1]
    pad = EPAD - E
    src_p = jnp.concatenate(
        [src, jnp.zeros((pad,), jnp.int32)]).reshape(NS, CPT, CHUNK)
    dst_p = jnp.concatenate(
        [dst, jnp.full((pad,), N, jnp.int32)]).reshape(NS, CPT, CHUNK)
    x_g = X_seq.reshape(G, N, FIN)

    deg = _deg_kernel(dst_p)
    hs1 = _xw_scale(x_g, W1, deg)
    acc1 = _conv_kernel(hs1, src_p, dst_p)
    hs2 = _mid(acc1.reshape(NC * N, PW), hs1, deg, b1.reshape(1, HG), W2)
    acc2 = _conv_kernel(hs2, src_p, dst_p)
    y = _final(acc2, hs2.reshape(NC, N, PW), deg, b2.reshape(1, HG),
               W_ih.T, W_hh.T, b_ih.reshape(1, 3 * HR),
               b_hh.reshape(1, 3 * HR), X_fut_seq.reshape(HOR, N, FNWP),
               Wd1[:HR], Wd1[HR:], bd1.reshape(1, HD), Wd2,
               bd2.reshape(1, 1))
    return y.reshape(1, HOR, N, 1)


# speedup vs baseline: 1.0440x; 1.0440x over previous
import functools

import jax
import jax.numpy as jnp
from jax import lax
from jax.experimental import pallas as pl
from jax.experimental.pallas import tpu as pltpu
from jax.experimental.pallas import tpu_sc as plsc

N = 10000
G = 8
E = 160000
FIN = 128
HG = 32
HR = 128
HD = 128
FNWP = 16
HOR = 4

NC = 2
NS = 16
GPC = G // NC
PW = GPC * HG

CHUNK = 128
CPT = 80
EPT = CHUNK * CPT
EPAD = EPT * NS
NPAD = N + 16
RT = 1000
NT_OUT = N // RT
RZ = NPAD // NS

HALF = CPT // 2
R = 4

NB = 5
RB = N // NB

_mesh = plsc.VectorSubcoreMesh(
    core_axis_name="c", subcore_axis_name="s", num_cores=NC, num_subcores=NS)

_sc_params = pltpu.CompilerParams(use_tc_tiling_on_sc=False)


@functools.partial(
    pl.kernel,
    out_type=jax.ShapeDtypeStruct((N, 16), jnp.float32),
    mesh=_mesh,
    compiler_params=_sc_params,
    scratch_types=[
        pltpu.VMEM((CPT, CHUNK), jnp.int32),
        pltpu.VMEM((CHUNK, 16), jnp.float32),
        pltpu.VMEM((RZ, 16), jnp.float32),
        pltpu.VMEM_SHARED((NPAD, 16), jnp.float32),
        pltpu.SemaphoreType.DMA,
    ],
)
def _deg_kernel(dst_hbm, deg_hbm, dst_v, ones_v, zero_v, deg_sh, sem):
    c = lax.axis_index("c")
    s = lax.axis_index("s")
    pltpu.sync_copy(dst_hbm.at[s], dst_v)

    def _fill(i, carry):
        ones_v[i, :] = jnp.ones((16,), jnp.float32)
        return carry
    lax.fori_loop(0, CHUNK, _fill, 0)

    def _zfill(i, carry):
        zero_v[i, :] = jnp.zeros((16,), jnp.float32)
        return carry
    lax.fori_loop(0, RZ, _zfill, 0)

    pltpu.sync_copy(zero_v, deg_sh.at[pl.ds(s * RZ, RZ)])
    plsc.subcore_barrier()

    DEPTH = 8
    for j in range(DEPTH):
        pltpu.async_copy(ones_v, deg_sh.at[dst_v.at[j]], sem, add=True)

    def _body(j, carry):
        pltpu.make_async_copy(ones_v, deg_sh.at[dst_v.at[j]], sem).wait()

        @pl.when(j + DEPTH < CPT)
        def _():
            pltpu.async_copy(
                ones_v, deg_sh.at[dst_v.at[j + DEPTH]], sem, add=True)
        return carry
    lax.fori_loop(0, CPT, _body, 0)
    plsc.subcore_barrier()

    @pl.when(jnp.logical_and(c == 0, s < NT_OUT))
    def _():
        pltpu.sync_copy(deg_sh.at[pl.ds(s * RT, RT)],
                        deg_hbm.at[pl.ds(s * RT, RT)])


@functools.partial(
    pl.kernel,
    out_type=jax.ShapeDtypeStruct((NC, N, PW), jnp.bfloat16),
    mesh=_mesh,
    compiler_params=_sc_params,
    scratch_types=[
        pltpu.VMEM((HALF, CHUNK), jnp.int32),
        pltpu.VMEM((HALF, CHUNK), jnp.int32),
        pltpu.VMEM((R, CHUNK, PW), jnp.bfloat16),
        pltpu.VMEM_SHARED((NPAD, PW), jnp.bfloat16),
        pltpu.SemaphoreType.DMA,
        pltpu.SemaphoreType.DMA,
        pltpu.SemaphoreType.DMA,
        pltpu.SemaphoreType.DMA,
        pltpu.SemaphoreType.DMA,
        pltpu.SemaphoreType.DMA,
        pltpu.SemaphoreType.DMA,
        pltpu.SemaphoreType.DMA,
    ],
)
def _conv_kernel(hs_hbm, src_hbm, dst_hbm, out_hbm,
                 dst_v, idx_v, rows_v, acc_sh,
                 sem_g0, sem_g1, sem_g2, sem_g3,
                 sem_s0, sem_s1, sem_s2, sem_s3):
    c = lax.axis_index("c")
    s = lax.axis_index("s")
    sem_g = (sem_g0, sem_g1, sem_g2, sem_g3)
    sem_s = (sem_s0, sem_s1, sem_s2, sem_s3)
    off = c * N

    def _zfill(i, carry):
        for k in range(PW // 32):
            rows_v[0, i, pl.ds(k * 32, 32)] = jnp.zeros((32,), jnp.bfloat16)
        return carry
    lax.fori_loop(0, CHUNK, _zfill, 0)
    for m in range(4):
        pltpu.sync_copy(rows_v.at[0],
                        acc_sh.at[pl.ds(s * RZ + m * CHUNK, CHUNK)])
    pltpu.sync_copy(rows_v.at[0, pl.ds(0, RZ - 4 * CHUNK)],
                    acc_sh.at[pl.ds(s * RZ + 4 * CHUNK, RZ - 4 * CHUNK)])
    plsc.subcore_barrier()

    def _gather(j, slot):
        pltpu.async_copy(
            hs_hbm.at[idx_v.at[j]], rows_v.at[slot], sem_g[slot])

    def _gather_wait(j, slot):
        pltpu.make_async_copy(
            hs_hbm.at[idx_v.at[j]], rows_v.at[slot], sem_g[slot]).wait()

    def _scatter(j, slot):
        pltpu.async_copy(
            rows_v.at[slot], acc_sh.at[dst_v.at[j]], sem_s[slot], add=True)

    def _scatter_wait(j, slot):
        pltpu.make_async_copy(
            rows_v.at[slot], acc_sh.at[dst_v.at[j]], sem_s[slot]).wait()

    for half in range(2):
        base = half * HALF
        pltpu.sync_copy(src_hbm.at[s, pl.ds(base, HALF)], idx_v)
        pltpu.sync_copy(dst_hbm.at[s, pl.ds(base, HALF)], dst_v)

        def _idxfill(j, c2):
            for k in range(CHUNK // 16):
                idx_v[j, pl.ds(k * 16, 16)] = (
                    idx_v[j, pl.ds(k * 16, 16)] + off)
            return c2
        lax.fori_loop(0, HALF, _idxfill, 0)

        for r in range(R - 1):
            _gather(r, r)

        def _grp_body(q, c2):
            for r in range(R):
                j = q * R + r
                nslot = (r + R - 1) % R
                _gather_wait(j, r)
                _scatter(j, r)
                @pl.when(j >= 1)
                def _():
                    _scatter_wait(j - 1, nslot)

                @pl.when(j + R - 1 < HALF)
                def _():
                    _gather(j + R - 1, nslot)
            return c2
        lax.fori_loop(0, HALF // R, _grp_body, 0)
        _scatter_wait(HALF - 1, R - 1)

    plsc.subcore_barrier()

    @pl.when(s < NT_OUT)
    def _():
        pltpu.sync_copy(acc_sh.at[pl.ds(s * RT, RT)],
                        out_hbm.at[c, pl.ds(s * RT, RT)])


def _xw_scale_body(x_ref, w_ref, deg_ref, o_ref):
    dinv = lax.rsqrt(deg_ref[:, 0:1] + 1.0)
    w = w_ref[...]
    parts = [
        jnp.dot(x_ref[gl], w, preferred_element_type=jnp.float32)
        for gl in range(GPC)
    ]
    o_ref[...] = (jnp.concatenate(parts, axis=1) * dinv).astype(jnp.bfloat16)


def _mid_body(acc_ref, hs_ref, deg_ref, b1_ref, w2_ref, o_ref):
    dinv = lax.rsqrt(deg_ref[:, 0:1] + 1.0)
    b1 = b1_ref[...]
    w2 = w2_ref[...]
    acc = acc_ref[...].astype(jnp.float32)
    hs = hs_ref[...].astype(jnp.float32)
    parts = []
    for gl in range(GPC):
        lo, hi = gl * HG, (gl + 1) * HG
        h1 = jnp.maximum(dinv * (acc[:, lo:hi] + hs[:, lo:hi]) + b1, 0.0)
        parts.append(jnp.dot(h1, w2, preferred_element_type=jnp.float32))
    o_ref[...] = (jnp.concatenate(parts, axis=1) * dinv).astype(jnp.bfloat16)


def _final_body(acc_ref, hs_ref, deg_ref, b2_ref, wih_ref, whh_ref,
                bih_ref, bhh_ref, xf_ref, wd1h_ref, wd1f_ref, bd1_ref,
                wd2_ref, bd2_ref, y_ref):
    dinv = lax.rsqrt(deg_ref[:, 0:1] + 1.0)
    b2 = b2_ref[...]
    wih = wih_ref[...]
    whh = whh_ref[...]
    bih = bih_ref[...]
    bhh = bhh_ref[...]
    h = jnp.zeros((RB, HR), jnp.float32)
    for t in range(G):
        cc, gl = t // GPC, t % GPC
        sl = pl.ds(gl * HG, HG)
        xt = dinv * (acc_ref[cc, :, sl].astype(jnp.float32)
                     + hs_ref[cc, :, sl].astype(jnp.float32)) + b2
        gi = jnp.dot(xt, wih, preferred_element_type=jnp.float32) + bih
        gh = jnp.dot(h, whh, preferred_element_type=jnp.float32) + bhh
        r = jax.nn.sigmoid(gi[:, 0:HR] + gh[:, 0:HR])
        z = jax.nn.sigmoid(gi[:, HR:2 * HR] + gh[:, HR:2 * HR])
        n = jnp.tanh(gi[:, 2 * HR:] + r * gh[:, 2 * HR:])
        h = (1.0 - z) * n + z * h
    hp = jnp.dot(h, wd1h_ref[...], preferred_element_type=jnp.float32)
    wd1f = wd1f_ref[...]
    bd1 = bd1_ref[...]
    wd2 = wd2_ref[...]
    bd2 = bd2_ref[...]
    for t in range(HOR):
        zz = jnp.maximum(
            hp + jnp.dot(xf_ref[t], wd1f, preferred_element_type=jnp.float32)
            + bd1, 0.0)
        y_ref[t] = jnp.dot(zz, wd2, preferred_element_type=jnp.float32) + bd2


def _xw_scale(x_g, w1, deg):
    return pl.pallas_call(
        _xw_scale_body,
        grid=(NC, NB),
        in_specs=[
            pl.BlockSpec((GPC, RB, FIN), lambda cb, nb: (cb, nb, 0)),
            pl.BlockSpec((FIN, HG), lambda cb, nb: (0, 0)),
            pl.BlockSpec((RB, 16), lambda cb, nb: (nb, 0)),
        ],
        out_specs=pl.BlockSpec((RB, PW), lambda cb, nb: (cb * NB + nb, 0)),
        out_shape=jax.ShapeDtypeStruct((NC * N, PW), jnp.bfloat16),
    )(x_g, w1, deg)


def _mid(acc1, hs1, deg, b1, w2):
    return pl.pallas_call(
        _mid_body,
        grid=(NC * NB,),
        in_specs=[
            pl.BlockSpec((RB, PW), lambda i: (i, 0)),
            pl.BlockSpec((RB, PW), lambda i: (i, 0)),
            pl.BlockSpec((RB, 16), lambda i: (i % NB, 0)),
            pl.BlockSpec((1, HG), lambda i: (0, 0)),
            pl.BlockSpec((HG, HG), lambda i: (0, 0)),
        ],
        out_specs=pl.BlockSpec((RB, PW), lambda i: (i, 0)),
        out_shape=jax.ShapeDtypeStruct((NC * N, PW), jnp.bfloat16),
    )(acc1, hs1, deg, b1, w2)


def _final(acc2, hs2, deg, b2, wih_t, whh_t, bih, bhh, xf, wd1h, wd1f,
           bd1, wd2, bd2):
    return pl.pallas_call(
        _final_body,
        grid=(NB,),
        in_specs=[
            pl.BlockSpec((NC, RB, PW), lambda i: (0, i, 0)),
            pl.BlockSpec((NC, RB, PW), lambda i: (0, i, 0)),
            pl.BlockSpec((RB, 16), lambda i: (i, 0)),
            pl.BlockSpec((1, HG), lambda i: (0, 0)),
            pl.BlockSpec((HG, 3 * HR), lambda i: (0, 0)),
            pl.BlockSpec((HR, 3 * HR), lambda i: (0, 0)),
            pl.BlockSpec((1, 3 * HR), lambda i: (0, 0)),
            pl.BlockSpec((1, 3 * HR), lambda i: (0, 0)),
            pl.BlockSpec((HOR, RB, FNWP), lambda i: (0, i, 0)),
            pl.BlockSpec((HR, HD), lambda i: (0, 0)),
            pl.BlockSpec((FNWP, HD), lambda i: (0, 0)),
            pl.BlockSpec((1, HD), lambda i: (0, 0)),
            pl.BlockSpec((HD, 1), lambda i: (0, 0)),
            pl.BlockSpec((1, 1), lambda i: (0, 0)),
        ],
        out_specs=pl.BlockSpec((HOR, RB, 1), lambda i: (0, i, 0)),
        out_shape=jax.ShapeDtypeStruct((HOR, N, 1), jnp.float32),
    )(acc2, hs2, deg, b2, wih_t, whh_t, bih, bhh, xf, wd1h, wd1f,
      bd1, wd2, bd2)


def kernel(X_seq, X_fut_seq, edge_index, W1, b1, W2, b2, W_ih, W_hh,
           b_ih, b_hh, Wd1, bd1, Wd2, bd2):
    src = edge_index[0]
    dst = edge_index[1]
    pad = EPAD - E
    src_p = jnp.concatenate(
        [src, jnp.zeros((pad,), jnp.int32)]).reshape(NS, CPT, CHUNK)
    dst_p = jnp.concatenate(
        [dst, jnp.full((pad,), N, jnp.int32)]).reshape(NS, CPT, CHUNK)
    x_g = X_seq.reshape(G, N, FIN)

    deg = _deg_kernel(dst_p)
    hs1 = _xw_scale(x_g, W1, deg)
    acc1 = _conv_kernel(hs1, src_p, dst_p)
    hs2 = _mid(acc1.reshape(NC * N, PW), hs1, deg, b1.reshape(1, HG), W2)
    acc2 = _conv_kernel(hs2, src_p, dst_p)
    y = _final(acc2, hs2.reshape(NC, N, PW), deg, b2.reshape(1, HG),
               W_ih.T, W_hh.T, b_ih.reshape(1, 3 * HR),
               b_hh.reshape(1, 3 * HR), X_fut_seq.reshape(HOR, N, FNWP),
               Wd1[:HR], Wd1[HR:], bd1.reshape(1, HD), Wd2,
               bd2.reshape(1, 1))
    return y.reshape(1, HOR, N, 1)
